# baseline (speedup 1.0000x reference)
import functools

import jax
import jax.numpy as jnp
from jax import lax
from jax.experimental import pallas as pl
from jax.experimental.pallas import tpu as pltpu
from jax.experimental.pallas import tpu_sc as plsc

N = 8192
D = 256
C = 32
NC = 2
NS = 16
NW = NC * NS
R = N // NW
L = 16
NCH = D // L
NB = 2
CPB = NCH // NB


def _sc_body(logt_hbm, emb_hbm, acc_hbm, cnt_hbm,
             logt_v, emb_v, cnt_v, asg_v, sem, *acc_refs):
    sid = lax.axis_index("s")
    cid = lax.axis_index("c")
    wid = sid * NC + cid
    base = wid * R

    emb_cp = pltpu.async_copy(emb_hbm.at[pl.ds(base, R)], emb_v, sem)
    pltpu.sync_copy(logt_hbm.at[:, pl.ds(base, R)], logt_v)

    zf = jnp.zeros((L,), jnp.float32)
    ones = jnp.ones((L,), jnp.float32)

    def zero_body(i, carry):
        for b in range(NB):
            for k in range(CPB):
                acc_refs[b][i, pl.ds(k * L, L)] = zf
        return carry

    lax.fori_loop(0, C, zero_body, 0)
    cnt_v[pl.ds(0, L)] = zf
    cnt_v[pl.ds(L, L)] = zf

    def am_body(g, carry):
        off = g * L
        m = logt_v[0, pl.ds(off, L)]
        a = jnp.zeros((L,), jnp.int32)
        for c in range(1, C):
            v = logt_v[c, pl.ds(off, L)]
            p = v > m
            m = jnp.where(p, v, m)
            a = jnp.where(p, jnp.full((L,), c, jnp.int32), a)
        asg_v[pl.ds(off, L)] = a
        plsc.addupdate_scatter(cnt_v, [a], ones)
        return carry

    lax.fori_loop(0, R // L, am_body, 0)

    emb_cp.wait()

    def grp_body(g, carry):
        avec = asg_v[pl.ds(g * L, L)]
        r0 = g * L
        for l in range(L):
            a = avec[l]
            for j in [0, 8, 1, 9, 2, 10, 3, 11, 4, 12, 5, 13, 6, 14, 7, 15]:
                plsc.addupdate(acc_refs[j // CPB].at[a, pl.ds((j % CPB) * L, L)],
                               emb_v[r0 + l, pl.ds(j * L, L)])
        return carry

    lax.fori_loop(0, R // L, grp_body, 0)

    for b in range(NB):
        pltpu.sync_copy(acc_refs[b],
                        acc_hbm.at[wid, :, pl.ds(b * CPB * L, CPB * L)])
    pltpu.sync_copy(cnt_v, cnt_hbm.at[wid])


@functools.cache
def _sc_partials():
    return pl.kernel(
        _sc_body,
        out_type=(
            jax.ShapeDtypeStruct((NW, C, D), jnp.float32),
            jax.ShapeDtypeStruct((NW, C), jnp.float32),
        ),
        mesh=plsc.VectorSubcoreMesh(core_axis_name="c", subcore_axis_name="s",
                                    num_cores=NC, num_subcores=NS),
        scratch_types=[
            pltpu.VMEM((C, R), jnp.float32),
            pltpu.VMEM((R, D), jnp.float32),
            pltpu.VMEM((C,), jnp.float32),
            pltpu.VMEM((R,), jnp.int32),
            pltpu.SemaphoreType.DMA,
        ] + [pltpu.VMEM((C, CPB * L), jnp.float32) for _ in range(NB)],
        compiler_params=pltpu.CompilerParams(needs_layout_passes=False),
    )


def _tc_body(acc_ref, cnt_ref, c_ref, o_ref, seg_ref):
    i = pl.program_id(0)

    @pl.when(i == 0)
    def _():
        seg_ref[...] = acc_ref[0]

    @pl.when(i > 0)
    def _():
        seg_ref[...] = seg_ref[...] + acc_ref[0]

    @pl.when(i == NW - 1)
    def _():
        w = jnp.sum(cnt_ref[...], axis=0)[:, None]
        inv = 1.0 / (w + 1e-8)
        m = (seg_ref[...] - w * c_ref[...]) * inv
        o_ref[...] = jnp.sqrt(jnp.sum(m * m, axis=1))


def kernel(embedding, centers, logits):
    logt = logits.T
    acc, cnt = _sc_partials()(logt, embedding)
    return pl.pallas_call(
        _tc_body,
        grid=(NW,),
        in_specs=[
            pl.BlockSpec((1, C, D), lambda i: (i, 0, 0)),
            pl.BlockSpec((NW, C), lambda i: (0, 0)),
            pl.BlockSpec((C, D), lambda i: (0, 0)),
        ],
        out_specs=pl.BlockSpec((C,), lambda i: (0,)),
        scratch_shapes=[pltpu.VMEM((C, D), jnp.float32)],
        out_shape=jax.ShapeDtypeStruct((C,), jnp.float32),
    )(acc, cnt, centers)


# speedup vs baseline: 1.2730x; 1.2730x over previous
import functools

import jax
import jax.numpy as jnp
from jax import lax
from jax.experimental import pallas as pl
from jax.experimental.pallas import tpu as pltpu
from jax.experimental.pallas import tpu_sc as plsc

N = 8192
D = 256
C = 32
NC = 2
NS = 16
NW = NC * NS
R = N // NW
L = 16
NCH = D // L
NB = 2
CPB = NCH // NB
_EPI_GRID = 4


def _sc_body(logt_hbm, emb_hbm, acc_hbm, cnt_hbm,
             logt_v, emb_v, cnt_v, asg_v, sem, *acc_refs):
    sid = lax.axis_index("s")
    cid = lax.axis_index("c")
    wid = sid * NC + cid
    base = wid * R

    emb_cp = pltpu.async_copy(emb_hbm.at[pl.ds(base, R)], emb_v, sem)
    pltpu.sync_copy(logt_hbm.at[:, pl.ds(base, R)], logt_v)

    zf = jnp.zeros((L,), jnp.float32)
    ones = jnp.ones((L,), jnp.float32)

    def zero_body(i, carry):
        for b in range(NB):
            for k in range(CPB):
                acc_refs[b][i, pl.ds(k * L, L)] = zf
        return carry

    lax.fori_loop(0, C, zero_body, 0)
    cnt_v[pl.ds(0, L)] = zf
    cnt_v[pl.ds(L, L)] = zf

    def am_body(g, carry):
        off = g * L
        m = logt_v[0, pl.ds(off, L)]
        a = jnp.zeros((L,), jnp.int32)
        for c in range(1, C):
            v = logt_v[c, pl.ds(off, L)]
            p = v > m
            m = jnp.where(p, v, m)
            a = jnp.where(p, jnp.full((L,), c, jnp.int32), a)
        asg_v[pl.ds(off, L)] = a
        plsc.addupdate_scatter(cnt_v, [a], ones)
        return carry

    lax.fori_loop(0, R // L, am_body, 0)

    emb_cp.wait()

    def grp_body(g, carry):
        avec = asg_v[pl.ds(g * L, L)]
        r0 = g * L
        for l in range(L):
            a = avec[l]
            for j in [0, 8, 1, 9, 2, 10, 3, 11, 4, 12, 5, 13, 6, 14, 7, 15]:
                plsc.addupdate(acc_refs[j // CPB].at[a, pl.ds((j % CPB) * L, L)],
                               emb_v[r0 + l, pl.ds(j * L, L)])
        return carry

    lax.fori_loop(0, R // L, grp_body, 0)

    for b in range(NB):
        pltpu.sync_copy(acc_refs[b],
                        acc_hbm.at[wid, :, pl.ds(b * CPB * L, CPB * L)])
    pltpu.sync_copy(cnt_v, cnt_hbm.at[wid])


@functools.cache
def _sc_partials():
    return pl.kernel(
        _sc_body,
        out_type=(
            jax.ShapeDtypeStruct((NW, C, D), jnp.float32),
            jax.ShapeDtypeStruct((NW, C), jnp.float32),
        ),
        mesh=plsc.VectorSubcoreMesh(core_axis_name="c", subcore_axis_name="s",
                                    num_cores=NC, num_subcores=NS),
        scratch_types=[
            pltpu.VMEM((C, R), jnp.float32),
            pltpu.VMEM((R, D), jnp.float32),
            pltpu.VMEM((C,), jnp.float32),
            pltpu.VMEM((R,), jnp.int32),
            pltpu.SemaphoreType.DMA,
        ] + [pltpu.VMEM((C, CPB * L), jnp.float32) for _ in range(NB)],
        compiler_params=pltpu.CompilerParams(needs_layout_passes=False),
    )


def _tc_body(acc_ref, cnt_ref, c_ref, o_ref, seg_ref):
    i = pl.program_id(0)

    part = jnp.sum(acc_ref[...], axis=0)

    @pl.when(i == 0)
    def _():
        seg_ref[...] = part

    @pl.when(i > 0)
    def _():
        seg_ref[...] = seg_ref[...] + part

    @pl.when(i == _EPI_GRID - 1)
    def _():
        w = jnp.sum(cnt_ref[...], axis=0)[:, None]
        inv = 1.0 / (w + 1e-8)
        m = (seg_ref[...] - w * c_ref[...]) * inv
        o_ref[...] = jnp.sqrt(jnp.sum(m * m, axis=1))


def kernel(embedding, centers, logits):
    logt = logits.T
    acc, cnt = _sc_partials()(logt, embedding)
    return pl.pallas_call(
        _tc_body,
        grid=(_EPI_GRID,),
        in_specs=[
            pl.BlockSpec((NW // _EPI_GRID, C, D), lambda i: (i, 0, 0)),
            pl.BlockSpec((NW, C), lambda i: (0, 0)),
            pl.BlockSpec((C, D), lambda i: (0, 0)),
        ],
        out_specs=pl.BlockSpec((C,), lambda i: (0,)),
        scratch_shapes=[pltpu.VMEM((C, D), jnp.float32)],
        out_shape=jax.ShapeDtypeStruct((C,), jnp.float32),
    )(acc, cnt, centers)


# speedup vs baseline: 1.6791x; 1.3191x over previous
import functools

import jax
import jax.numpy as jnp
from jax import lax
from jax.experimental import pallas as pl
from jax.experimental.pallas import tpu as pltpu
from jax.experimental.pallas import tpu_sc as plsc

N = 8192
D = 256
C = 32
NC = 2
NS = 16
NW = NC * NS
R = N // NW
L = 16
NCH = D // L
NB = 2
CPB = NCH // NB


def _sc_body(logt_hbm, emb_hbm, acc_hbm, cnt_hbm,
             logt_v, emb_v, cnt_v, asg_v, sem, *acc_refs):
    sid = lax.axis_index("s")
    cid = lax.axis_index("c")
    wid = sid * NC + cid
    base = wid * R

    emb_cp = pltpu.async_copy(emb_hbm.at[pl.ds(base, R)], emb_v, sem)
    pltpu.sync_copy(logt_hbm.at[:, pl.ds(base, R)], logt_v)

    zf = jnp.zeros((L,), jnp.float32)
    ones = jnp.ones((L,), jnp.float32)

    def zero_body(i, carry):
        for b in range(NB):
            for k in range(CPB):
                acc_refs[b][i, pl.ds(k * L, L)] = zf
        return carry

    lax.fori_loop(0, C, zero_body, 0)
    cnt_v[pl.ds(0, L)] = zf
    cnt_v[pl.ds(L, L)] = zf

    def am_body(g, carry):
        off = g * L
        m = logt_v[0, pl.ds(off, L)]
        a = jnp.zeros((L,), jnp.int32)
        for c in range(1, C):
            v = logt_v[c, pl.ds(off, L)]
            p = v > m
            m = jnp.where(p, v, m)
            a = jnp.where(p, jnp.full((L,), c, jnp.int32), a)
        asg_v[pl.ds(off, L)] = a
        plsc.addupdate_scatter(cnt_v, [a], ones)
        return carry

    lax.fori_loop(0, R // L, am_body, 0)

    emb_cp.wait()

    def grp_body(g, carry):
        avec = asg_v[pl.ds(g * L, L)]
        r0 = g * L
        for l in range(L):
            a = avec[l]
            vals = [emb_v[r0 + l, pl.ds(j * L, L)] for j in range(NCH)]
            for j in [0, 8, 1, 9, 2, 10, 3, 11, 4, 12, 5, 13, 6, 14, 7, 15]:
                plsc.addupdate(acc_refs[j // CPB].at[a, pl.ds((j % CPB) * L, L)],
                               vals[j])
        return carry

    lax.fori_loop(0, R // L, grp_body, 0)

    for b in range(NB):
        pltpu.sync_copy(acc_refs[b],
                        acc_hbm.at[wid, :, pl.ds(b * CPB * L, CPB * L)])
    pltpu.sync_copy(cnt_v, cnt_hbm.at[wid])


@functools.cache
def _sc_partials():
    return pl.kernel(
        _sc_body,
        out_type=(
            jax.ShapeDtypeStruct((NW, C, D), jnp.float32),
            jax.ShapeDtypeStruct((NW, C), jnp.float32),
        ),
        mesh=plsc.VectorSubcoreMesh(core_axis_name="c", subcore_axis_name="s",
                                    num_cores=NC, num_subcores=NS),
        scratch_types=[
            pltpu.VMEM((C, R), jnp.float32),
            pltpu.VMEM((R, D), jnp.float32),
            pltpu.VMEM((C,), jnp.float32),
            pltpu.VMEM((R,), jnp.int32),
            pltpu.SemaphoreType.DMA,
        ] + [pltpu.VMEM((C, CPB * L), jnp.float32) for _ in range(NB)],
        compiler_params=pltpu.CompilerParams(needs_layout_passes=False),
    )


def _tc_body(acc_ref, cnt_ref, c_ref, o_ref):
    w = jnp.sum(cnt_ref[...], axis=0)[:, None]
    inv = 1.0 / (w + 1e-8)
    seg = jnp.sum(acc_ref[...], axis=0)
    m = (seg - w * c_ref[...]) * inv
    o_ref[...] = jnp.sqrt(jnp.sum(m * m, axis=1))


def kernel(embedding, centers, logits):
    logt = logits.T
    acc, cnt = _sc_partials()(logt, embedding)
    return pl.pallas_call(
        _tc_body,
        out_shape=jax.ShapeDtypeStruct((C,), jnp.float32),
    )(acc, cnt, centers)


# speedup vs baseline: 1.6819x; 1.0017x over previous
import functools

import jax
import jax.numpy as jnp
from jax import lax
from jax.experimental import pallas as pl
from jax.experimental.pallas import tpu as pltpu
from jax.experimental.pallas import tpu_sc as plsc

N = 8192
D = 256
C = 32
NC = 2
NS = 16
NW = NC * NS
R = N // NW
L = 16
NCH = D // L
NB = 2
CPB = NCH // NB


def _sc_body(logt_hbm, emb_hbm, acc_hbm, cnt_hbm,
             logt_v, emb_v, cnt_v, asg_v, sem, *acc_refs):
    sid = lax.axis_index("s")
    cid = lax.axis_index("c")
    wid = sid * NC + cid
    base = wid * R

    emb_cp = pltpu.async_copy(emb_hbm.at[pl.ds(base, R)], emb_v, sem)
    pltpu.sync_copy(logt_hbm.at[:, pl.ds(base, R)], logt_v)

    zf = jnp.zeros((L,), jnp.float32)
    ones = jnp.ones((L,), jnp.float32)

    def zero_body(i, carry):
        for b in range(NB):
            for k in range(CPB):
                acc_refs[b][i, pl.ds(k * L, L)] = zf
        return carry

    lax.fori_loop(0, C, zero_body, 0)
    cnt_v[pl.ds(0, L)] = zf
    cnt_v[pl.ds(L, L)] = zf

    def am_body(g, carry):
        off = g * L
        m = logt_v[0, pl.ds(off, L)]
        a = jnp.zeros((L,), jnp.int32)
        for c in range(1, C):
            v = logt_v[c, pl.ds(off, L)]
            p = v > m
            m = jnp.where(p, v, m)
            a = jnp.where(p, jnp.full((L,), c, jnp.int32), a)
        asg_v[pl.ds(off, L)] = a
        plsc.addupdate_scatter(cnt_v, [a], ones)
        return carry

    lax.fori_loop(0, R // L, am_body, 0)

    emb_cp.wait()

    st_order = [0, 8, 1, 9, 2, 10, 3, 11, 4, 12, 5, 13, 6, 14, 7, 15]

    def _row_stores(a, vals):
        for j in st_order:
            plsc.addupdate(acc_refs[j // CPB].at[a, pl.ds((j % CPB) * L, L)],
                           vals[j])

    def grp_body(g, carry):
        avec = asg_v[pl.ds(g * L, L)]
        r0 = g * L
        prev_a = avec[0]
        prev_vals = [emb_v[r0, pl.ds(j * L, L)] for j in range(NCH)]
        for l in range(1, L):
            a = avec[l]
            vals = [None] * NCH
            for k in range(NCH):
                vals[k] = emb_v[r0 + l, pl.ds(k * L, L)]
                j = st_order[k]
                plsc.addupdate(acc_refs[j // CPB].at[prev_a, pl.ds((j % CPB) * L, L)],
                               prev_vals[j])
            prev_a, prev_vals = a, vals
        _row_stores(prev_a, prev_vals)
        return carry

    lax.fori_loop(0, R // L, grp_body, 0)

    for b in range(NB):
        pltpu.sync_copy(acc_refs[b],
                        acc_hbm.at[wid, :, pl.ds(b * CPB * L, CPB * L)])
    pltpu.sync_copy(cnt_v, cnt_hbm.at[wid])


@functools.cache
def _sc_partials():
    return pl.kernel(
        _sc_body,
        out_type=(
            jax.ShapeDtypeStruct((NW, C, D), jnp.float32),
            jax.ShapeDtypeStruct((NW, C), jnp.float32),
        ),
        mesh=plsc.VectorSubcoreMesh(core_axis_name="c", subcore_axis_name="s",
                                    num_cores=NC, num_subcores=NS),
        scratch_types=[
            pltpu.VMEM((C, R), jnp.float32),
            pltpu.VMEM((R, D), jnp.float32),
            pltpu.VMEM((C,), jnp.float32),
            pltpu.VMEM((R,), jnp.int32),
            pltpu.SemaphoreType.DMA,
        ] + [pltpu.VMEM((C, CPB * L), jnp.float32) for _ in range(NB)],
        compiler_params=pltpu.CompilerParams(needs_layout_passes=False),
    )


def _tc_body(acc_ref, cnt_ref, c_ref, o_ref):
    w = jnp.sum(cnt_ref[...], axis=0)[:, None]
    inv = 1.0 / (w + 1e-8)
    seg = jnp.sum(acc_ref[...], axis=0)
    m = (seg - w * c_ref[...]) * inv
    o_ref[...] = jnp.sqrt(jnp.sum(m * m, axis=1))


def kernel(embedding, centers, logits):
    logt = logits.T
    acc, cnt = _sc_partials()(logt, embedding)
    return pl.pallas_call(
        _tc_body,
        out_shape=jax.ShapeDtypeStruct((C,), jnp.float32),
    )(acc, cnt, centers)
